# baseline (speedup 1.0000x reference)
import jax
import jax.numpy as jnp
from jax.experimental import pallas as pl
from jax.experimental.pallas import tpu as pltpu
from jax.experimental.pallas import tpu_sc as plsc

_N_TOKEN = 1000000
_D_PROJ = 128
_EMB_SCALE = float(_D_PROJ) ** 0.5
_P_ROWS = 1008000

_GATHER_WINDOW = 128


def _project_b0(emb0, p0t, rows):
    blk = 5000

    def body(e_ref, p_ref, o_ref):
        o_ref[...] = jnp.dot(
            e_ref[...], p_ref[...], preferred_element_type=jnp.float32
        )

    return pl.pallas_call(
        body,
        grid=(100000 // blk,),
        in_specs=[
            pl.BlockSpec((blk, 128), lambda i: (i, 0)),
            pl.BlockSpec((128, 128), lambda i: (0, 0)),
        ],
        out_specs=pl.BlockSpec((blk, 128), lambda i: (i, 0)),
        out_shape=jax.ShapeDtypeStruct((rows, _D_PROJ), jnp.float32),
    )(emb0, p0t)


def _project_packed(embp, wbig, p_buf, *, factor, base, n_chunks):
    blk_out = 1000 * factor

    def body(e_ref, w_ref, b_ref, o_ref):
        del b_ref
        for j in range(factor):
            o_ref[pl.ds(1000 * j, 1000), :] = jnp.dot(
                e_ref[...], w_ref[j], preferred_element_type=jnp.float32
            )

    return pl.pallas_call(
        body,
        grid=(n_chunks,),
        in_specs=[
            pl.BlockSpec((1000, 128), lambda i: (i, 0)),
            pl.BlockSpec((factor, 128, 128), lambda i: (0, 0, 0)),
            pl.BlockSpec(memory_space=pl.ANY),
        ],
        out_specs=pl.BlockSpec(
            (blk_out, 128), lambda i: (base // blk_out + i, 0)
        ),
        out_shape=jax.ShapeDtypeStruct((_P_ROWS, _D_PROJ), jnp.float32),
        input_output_aliases={2: 0},
    )(embp, wbig, p_buf)


def _transform_idx(inp):
    def body(t_ref, o_ref):
        t = t_ref[...]
        u1 = t - 100000
        r1 = u1 >> 2
        c1 = (r1.astype(jnp.float32) * jnp.float32(1.0 / 1000.0)).astype(jnp.int32)
        i1 = 100000 + 4000 * c1 + 1000 * (u1 & 3) + (r1 - 1000 * c1)
        u2 = t - 400000
        r2 = u2 >> 4
        c2 = (r2.astype(jnp.float32) * jnp.float32(1.0 / 1000.0)).astype(jnp.int32)
        i2 = 400000 + 16000 * c2 + 1000 * (u2 & 15) + (r2 - 1000 * c2)
        o_ref[...] = jnp.where(t < 100000, t, jnp.where(t < 400000, i1, i2))

    shape = inp.shape

    return pl.pallas_call(
        body,
        grid=(8,),
        in_specs=[pl.BlockSpec((shape[0] // 8, shape[1]), lambda i: (i, 0))],
        out_specs=pl.BlockSpec((shape[0] // 8, shape[1]), lambda i: (i, 0)),
        out_shape=jax.ShapeDtypeStruct(shape, jnp.int32),
    )(inp)


def _sc_gather(table, idx_flat, n):
    mesh = plsc.VectorSubcoreMesh(core_axis_name="core", subcore_axis_name="subcore")

    @pl.kernel(
        out_type=jax.ShapeDtypeStruct((n, _D_PROJ), jnp.float32),
        mesh=mesh,
    )
    def k(tbl_hbm, i_hbm, o_hbm):
        def body(i_vmem, o_vmem):
            pltpu.sync_copy(tbl_hbm.at[i_vmem.at[0]], o_vmem)

        pltpu.emit_pipeline(
            body,
            grid=(n // _GATHER_WINDOW,),
            in_specs=[
                pl.BlockSpec((1, _GATHER_WINDOW), lambda i: (0, i)),
            ],
            out_specs=[
                pl.BlockSpec((_GATHER_WINDOW, _D_PROJ), lambda i: (i, 0)),
            ],
            core_axis_name=("core", "subcore"),
            dimension_semantics=(pltpu.PARALLEL,),
        )(i_hbm, o_hbm)

    return k(table, idx_flat)


def _expand_weight(pt, factor):
    d = pt.shape[0]
    w = jnp.zeros((factor, 128, 128), jnp.float32)
    for j in range(factor):
        w = w.at[j, j * d:(j + 1) * d, :].set(pt)
    return w


def kernel(inp, emb0, emb1, emb2, proj0, proj1, proj2):
    scale = jnp.float32(_EMB_SCALE)
    p0t = proj0.T * scale
    w1 = _expand_weight(proj1.T * scale, 4)
    w2 = _expand_weight(proj2.T * scale, 16)
    emb1p = emb1.reshape(75000, 128)
    emb2p = jnp.pad(emb2.reshape(37500, 128), ((0, 500), (0, 0)))
    p_buf = _project_b0(emb0, p0t, _P_ROWS)
    p_buf = _project_packed(emb1p, w1, p_buf, factor=4, base=100000, n_chunks=75)
    p_buf = _project_packed(emb2p, w2, p_buf, factor=16, base=400000, n_chunks=38)
    idx = _transform_idx(inp)
    idx_flat = idx.reshape(1, -1)
    n = idx_flat.shape[1]
    out = _sc_gather(p_buf, idx_flat, n)
    return out.reshape(inp.shape + (_D_PROJ,))


# speedup vs baseline: 1.4082x; 1.4082x over previous
import jax
import jax.numpy as jnp
from jax import lax
from jax.experimental import pallas as pl
from jax.experimental.pallas import tpu as pltpu
from jax.experimental.pallas import tpu_sc as plsc

_D_PROJ = 128
_EMB_SCALE = float(_D_PROJ) ** 0.5

_S1 = 102400
_S2 = 405504
_P_ROWS = 1007616

_GATHER_WINDOW = 128

_TDIMS = (((0,), (0,)), ((), ()))


def _project_b0(emb0, p0t, rows):
    blk = 5000

    def body(e_ref, p_ref, o_ref):
        o_ref[...] = jnp.dot(
            e_ref[...], p_ref[...], preferred_element_type=jnp.float32
        )

    return pl.pallas_call(
        body,
        grid=(100000 // blk,),
        in_specs=[
            pl.BlockSpec((blk, 128), lambda i: (i, 0)),
            pl.BlockSpec((128, 128), lambda i: (0, 0)),
        ],
        out_specs=pl.BlockSpec((blk, 128), lambda i: (i, 0)),
        out_shape=jax.ShapeDtypeStruct((rows, _D_PROJ), jnp.float32),
    )(emb0, p0t)


def _project_t(embt, pt, p_buf, *, base, n_chunks):
    d = embt.shape[0]
    blk = 4096

    def body(e_ref, p_ref, b_ref, o_ref):
        del b_ref
        o_ref[...] = lax.dot_general(
            e_ref[...], p_ref[...], _TDIMS, preferred_element_type=jnp.float32
        )

    return pl.pallas_call(
        body,
        grid=(n_chunks,),
        in_specs=[
            pl.BlockSpec((d, blk), lambda i: (0, i)),
            pl.BlockSpec((d, 128), lambda i: (0, 0)),
            pl.BlockSpec(memory_space=pl.ANY),
        ],
        out_specs=pl.BlockSpec((blk, 128), lambda i: (base // blk + i, 0)),
        out_shape=jax.ShapeDtypeStruct((_P_ROWS, _D_PROJ), jnp.float32),
        input_output_aliases={2: 0},
    )(embt, pt, p_buf)


def _transform_idx(inp):
    def body(t_ref, o_ref):
        t = t_ref[...]
        o_ref[...] = t + jnp.where(
            t < 100000, 0, jnp.where(t < 400000, _S1 - 100000, _S2 - 400000)
        )

    shape = inp.shape

    return pl.pallas_call(
        body,
        grid=(8,),
        in_specs=[pl.BlockSpec((shape[0] // 8, shape[1]), lambda i: (i, 0))],
        out_specs=pl.BlockSpec((shape[0] // 8, shape[1]), lambda i: (i, 0)),
        out_shape=jax.ShapeDtypeStruct(shape, jnp.int32),
    )(inp)


def _sc_gather(table, idx_flat, n):
    mesh = plsc.VectorSubcoreMesh(core_axis_name="core", subcore_axis_name="subcore")

    @pl.kernel(
        out_type=jax.ShapeDtypeStruct((n, _D_PROJ), jnp.float32),
        mesh=mesh,
    )
    def k(tbl_hbm, i_hbm, o_hbm):
        def body(i_vmem, o_vmem):
            pltpu.sync_copy(tbl_hbm.at[i_vmem.at[0]], o_vmem)

        pltpu.emit_pipeline(
            body,
            grid=(n // _GATHER_WINDOW,),
            in_specs=[
                pl.BlockSpec((1, _GATHER_WINDOW), lambda i: (0, i)),
            ],
            out_specs=[
                pl.BlockSpec((_GATHER_WINDOW, _D_PROJ), lambda i: (i, 0)),
            ],
            core_axis_name=("core", "subcore"),
            dimension_semantics=(pltpu.PARALLEL,),
        )(i_hbm, o_hbm)

    return k(table, idx_flat)


def kernel(inp, emb0, emb1, emb2, proj0, proj1, proj2):
    scale = jnp.float32(_EMB_SCALE)
    p0t = proj0.T * scale
    p1t = proj1.T * scale
    p2t = proj2.T * scale
    p_buf = _project_b0(emb0, p0t, _P_ROWS)
    p_buf = _project_t(emb1.T, p1t, p_buf, base=_S1, n_chunks=74)
    p_buf = _project_t(emb2.T, p2t, p_buf, base=_S2, n_chunks=147)
    idx = _transform_idx(inp)
    idx_flat = idx.reshape(1, -1)
    n = idx_flat.shape[1]
    out = _sc_gather(p_buf, idx_flat, n)
    return out.reshape(inp.shape + (_D_PROJ,))


# speedup vs baseline: 1.5520x; 1.1021x over previous
import jax
import jax.numpy as jnp
from jax import lax
from jax.experimental import pallas as pl
from jax.experimental.pallas import tpu as pltpu
from jax.experimental.pallas import tpu_sc as plsc

_D_PROJ = 128
_EMB_SCALE = float(_D_PROJ) ** 0.5

_S1 = 106496
_S2 = 409600
_P_ROWS = 1015808

_GATHER_WINDOW = 128

_TDIMS = (((0,), (0,)), ((), ()))


def _project_b0(emb0, p0t, rows):
    blk = 8192

    def body(e_ref, p_ref, o_ref):
        o_ref[...] = jnp.dot(
            e_ref[...], p_ref[...], preferred_element_type=jnp.float32
        )

    return pl.pallas_call(
        body,
        grid=(13,),
        in_specs=[
            pl.BlockSpec((blk, 128), lambda i: (i, 0)),
            pl.BlockSpec((128, 128), lambda i: (0, 0)),
        ],
        out_specs=pl.BlockSpec((blk, 128), lambda i: (i, 0)),
        out_shape=jax.ShapeDtypeStruct((rows, _D_PROJ), jnp.float32),
    )(emb0, p0t)


def _project_t(embt, pt, p_buf, *, base, n_chunks):
    d = embt.shape[0]
    blk = 8192

    def body(e_ref, p_ref, b_ref, o_ref):
        del b_ref
        o_ref[...] = lax.dot_general(
            e_ref[...], p_ref[...], _TDIMS, preferred_element_type=jnp.float32
        )

    return pl.pallas_call(
        body,
        grid=(n_chunks,),
        in_specs=[
            pl.BlockSpec((d, blk), lambda i: (0, i)),
            pl.BlockSpec((d, 128), lambda i: (0, 0)),
            pl.BlockSpec(memory_space=pl.ANY),
        ],
        out_specs=pl.BlockSpec((blk, 128), lambda i: (base // blk + i, 0)),
        out_shape=jax.ShapeDtypeStruct((_P_ROWS, _D_PROJ), jnp.float32),
        input_output_aliases={2: 0},
    )(embt, pt, p_buf)


def _transform_idx(inp):
    def body(t_ref, o_ref):
        t = t_ref[...]
        o_ref[...] = t + jnp.where(
            t < 100000, 0, jnp.where(t < 400000, _S1 - 100000, _S2 - 400000)
        )

    shape = inp.shape

    return pl.pallas_call(
        body,
        grid=(8,),
        in_specs=[pl.BlockSpec((shape[0] // 8, shape[1]), lambda i: (i, 0))],
        out_specs=pl.BlockSpec((shape[0] // 8, shape[1]), lambda i: (i, 0)),
        out_shape=jax.ShapeDtypeStruct(shape, jnp.int32),
    )(inp)


def _sc_gather(table, idx_flat, n):
    mesh = plsc.VectorSubcoreMesh(core_axis_name="core", subcore_axis_name="subcore")

    @pl.kernel(
        out_type=jax.ShapeDtypeStruct((n, _D_PROJ), jnp.float32),
        mesh=mesh,
    )
    def k(tbl_hbm, i_hbm, o_hbm):
        def body(i_vmem, o_vmem):
            pltpu.sync_copy(tbl_hbm.at[i_vmem.at[0]], o_vmem)

        pltpu.emit_pipeline(
            body,
            grid=(n // _GATHER_WINDOW,),
            in_specs=[
                pl.BlockSpec((1, _GATHER_WINDOW), lambda i: (0, i)),
            ],
            out_specs=[
                pl.BlockSpec((_GATHER_WINDOW, _D_PROJ), lambda i: (i, 0)),
            ],
            core_axis_name=("core", "subcore"),
            dimension_semantics=(pltpu.PARALLEL,),
        )(i_hbm, o_hbm)

    return k(table, idx_flat)


def kernel(inp, emb0, emb1, emb2, proj0, proj1, proj2):
    scale = jnp.float32(_EMB_SCALE)
    p0t = proj0.T * scale
    p1t = proj1.T * scale
    p2t = proj2.T * scale
    p_buf = _project_b0(emb0, p0t, _P_ROWS)
    p_buf = _project_t(emb1.T, p1t, p_buf, base=_S1, n_chunks=37)
    p_buf = _project_t(emb2.T, p2t, p_buf, base=_S2, n_chunks=74)
    idx = _transform_idx(inp)
    idx_flat = idx.reshape(1, -1)
    n = idx_flat.shape[1]
    out = _sc_gather(p_buf, idx_flat, n)
    return out.reshape(inp.shape + (_D_PROJ,))


# speedup vs baseline: 1.5577x; 1.0037x over previous
import jax
import jax.numpy as jnp
from jax import lax
from jax.experimental import pallas as pl
from jax.experimental.pallas import tpu as pltpu
from jax.experimental.pallas import tpu_sc as plsc

_D_PROJ = 128
_EMB_SCALE = float(_D_PROJ) ** 0.5

_BLK = 8192
_S1 = 106496
_S2 = 409600
_P_ROWS = 1015808
_B0_BLOCKS = _S1 // _BLK
_B1_BLOCKS = (_S2 - _S1) // _BLK
_B2_BLOCKS = (_P_ROWS - _S2) // _BLK

_GATHER_WINDOW = 128

_TDIMS = (((0,), (0,)), ((), ()))


def _project_tables(emb0, emb1t, emb2t, p0t, p1t, p2t):
    def body(e0_ref, e1_ref, e2_ref, p0_ref, p1_ref, p2_ref, out_ref):
        i = pl.program_id(0)

        @pl.when(i < _B0_BLOCKS)
        def _():
            out_ref[...] = jnp.dot(
                e0_ref[...], p0_ref[...], preferred_element_type=jnp.float32
            )

        @pl.when(jnp.logical_and(i >= _B0_BLOCKS, i < _B0_BLOCKS + _B1_BLOCKS))
        def _():
            out_ref[...] = lax.dot_general(
                e1_ref[...], p1_ref[...], _TDIMS,
                preferred_element_type=jnp.float32,
            )

        @pl.when(i >= _B0_BLOCKS + _B1_BLOCKS)
        def _():
            out_ref[...] = lax.dot_general(
                e2_ref[...], p2_ref[...], _TDIMS,
                preferred_element_type=jnp.float32,
            )

    return pl.pallas_call(
        body,
        grid=(_B0_BLOCKS + _B1_BLOCKS + _B2_BLOCKS,),
        in_specs=[
            pl.BlockSpec(
                (_BLK, 128), lambda i: (jnp.minimum(i, _B0_BLOCKS - 1), 0)
            ),
            pl.BlockSpec(
                (32, _BLK),
                lambda i: (0, jnp.clip(i - _B0_BLOCKS, 0, _B1_BLOCKS - 1)),
            ),
            pl.BlockSpec(
                (8, _BLK),
                lambda i: (
                    0,
                    jnp.clip(i - _B0_BLOCKS - _B1_BLOCKS, 0, _B2_BLOCKS - 1),
                ),
            ),
            pl.BlockSpec((128, 128), lambda i: (0, 0)),
            pl.BlockSpec((32, 128), lambda i: (0, 0)),
            pl.BlockSpec((8, 128), lambda i: (0, 0)),
        ],
        out_specs=pl.BlockSpec((_BLK, 128), lambda i: (i, 0)),
        out_shape=jax.ShapeDtypeStruct((_P_ROWS, _D_PROJ), jnp.float32),
        compiler_params=pltpu.CompilerParams(
            dimension_semantics=("parallel",),
        ),
    )(emb0, emb1t, emb2t, p0t, p1t, p2t)


def _transform_idx(inp):
    def body(t_ref, o_ref):
        t = t_ref[...]
        o_ref[...] = t + jnp.where(
            t < 100000, 0, jnp.where(t < 400000, _S1 - 100000, _S2 - 400000)
        )

    shape = inp.shape

    return pl.pallas_call(
        body,
        grid=(8,),
        in_specs=[pl.BlockSpec((shape[0] // 8, shape[1]), lambda i: (i, 0))],
        out_specs=pl.BlockSpec((shape[0] // 8, shape[1]), lambda i: (i, 0)),
        out_shape=jax.ShapeDtypeStruct(shape, jnp.int32),
    )(inp)


def _sc_gather(table, idx_flat, n):
    mesh = plsc.VectorSubcoreMesh(core_axis_name="core", subcore_axis_name="subcore")

    @pl.kernel(
        out_type=jax.ShapeDtypeStruct((n, _D_PROJ), jnp.float32),
        mesh=mesh,
    )
    def k(tbl_hbm, i_hbm, o_hbm):
        def body(i_vmem, o_vmem):
            pltpu.sync_copy(tbl_hbm.at[i_vmem.at[0]], o_vmem)

        pltpu.emit_pipeline(
            body,
            grid=(n // _GATHER_WINDOW,),
            in_specs=[
                pl.BlockSpec((1, _GATHER_WINDOW), lambda i: (0, i)),
            ],
            out_specs=[
                pl.BlockSpec((_GATHER_WINDOW, _D_PROJ), lambda i: (i, 0)),
            ],
            core_axis_name=("core", "subcore"),
            dimension_semantics=(pltpu.PARALLEL,),
        )(i_hbm, o_hbm)

    return k(table, idx_flat)


def kernel(inp, emb0, emb1, emb2, proj0, proj1, proj2):
    scale = jnp.float32(_EMB_SCALE)
    p0t = proj0.T * scale
    p1t = proj1.T * scale
    p2t = proj2.T * scale
    table = _project_tables(emb0, emb1.T, emb2.T, p0t, p1t, p2t)
    idx = _transform_idx(inp)
    idx_flat = idx.reshape(1, -1)
    n = idx_flat.shape[1]
    out = _sc_gather(table, idx_flat, n)
    return out.reshape(inp.shape + (_D_PROJ,))


# speedup vs baseline: 1.6216x; 1.0410x over previous
import jax
import jax.numpy as jnp
from jax import lax
from jax.experimental import pallas as pl
from jax.experimental.pallas import tpu as pltpu
from jax.experimental.pallas import tpu_sc as plsc

_D_PROJ = 128
_EMB_SCALE = float(_D_PROJ) ** 0.5

_BLK = 8192
_S1 = 106496
_S2 = 409600
_P_ROWS = 1015808
_B0_BLOCKS = _S1 // _BLK
_B1_BLOCKS = (_S2 - _S1) // _BLK
_B2_BLOCKS = (_P_ROWS - _S2) // _BLK

_GATHER_WINDOW = 128

_TDIMS = (((0,), (0,)), ((), ()))


def _project_tables(emb0, emb1t, emb2t, p0t, p1t, p2t):
    def body(e0_ref, e1_ref, e2_ref, p0_ref, p1_ref, p2_ref, out_ref):
        i = pl.program_id(0)

        @pl.when(i < _B0_BLOCKS)
        def _():
            out_ref[...] = jnp.dot(
                e0_ref[...], p0_ref[...], preferred_element_type=jnp.float32
            )

        @pl.when(jnp.logical_and(i >= _B0_BLOCKS, i < _B0_BLOCKS + _B1_BLOCKS))
        def _():
            out_ref[...] = lax.dot_general(
                e1_ref[...], p1_ref[...], _TDIMS,
                preferred_element_type=jnp.float32,
            )

        @pl.when(i >= _B0_BLOCKS + _B1_BLOCKS)
        def _():
            out_ref[...] = lax.dot_general(
                e2_ref[...], p2_ref[...], _TDIMS,
                preferred_element_type=jnp.float32,
            )

    return pl.pallas_call(
        body,
        grid=(_B0_BLOCKS + _B1_BLOCKS + _B2_BLOCKS,),
        in_specs=[
            pl.BlockSpec(
                (_BLK, 128), lambda i: (jnp.minimum(i, _B0_BLOCKS - 1), 0)
            ),
            pl.BlockSpec(
                (32, _BLK),
                lambda i: (0, jnp.clip(i - _B0_BLOCKS, 0, _B1_BLOCKS - 1)),
            ),
            pl.BlockSpec(
                (8, _BLK),
                lambda i: (
                    0,
                    jnp.clip(i - _B0_BLOCKS - _B1_BLOCKS, 0, _B2_BLOCKS - 1),
                ),
            ),
            pl.BlockSpec((128, 128), lambda i: (0, 0)),
            pl.BlockSpec((32, 128), lambda i: (0, 0)),
            pl.BlockSpec((8, 128), lambda i: (0, 0)),
        ],
        out_specs=pl.BlockSpec((_BLK, 128), lambda i: (i, 0)),
        out_shape=jax.ShapeDtypeStruct((_P_ROWS, _D_PROJ), jnp.float32),
        compiler_params=pltpu.CompilerParams(
            dimension_semantics=("parallel",),
        ),
    )(emb0, emb1t, emb2t, p0t, p1t, p2t)


def _transform_idx(inp):
    def body(t_ref, o_ref):
        t = t_ref[...]
        o_ref[...] = t + jnp.where(
            t < 100000, 0, jnp.where(t < 400000, _S1 - 100000, _S2 - 400000)
        )

    shape = inp.shape

    return pl.pallas_call(
        body,
        grid=(8,),
        in_specs=[pl.BlockSpec((shape[0] // 8, shape[1]), lambda i: (i, 0))],
        out_specs=pl.BlockSpec((shape[0] // 8, shape[1]), lambda i: (i, 0)),
        out_shape=jax.ShapeDtypeStruct(shape, jnp.int32),
    )(inp)


def _sc_gather(table, idx_flat, n):
    mesh = plsc.VectorSubcoreMesh(core_axis_name="core", subcore_axis_name="subcore")

    @pl.kernel(
        out_type=jax.ShapeDtypeStruct((n, _D_PROJ), jnp.float32),
        mesh=mesh,
    )
    def k(tbl_hbm, i_hbm, o_hbm):
        def body(i_vmem, o_vmem):
            pltpu.sync_copy(
                tbl_hbm.at[i_vmem.at[0, pl.ds(0, _GATHER_WINDOW)]],
                o_vmem.at[pl.ds(0, _GATHER_WINDOW)],
            )
            pltpu.sync_copy(
                tbl_hbm.at[i_vmem.at[0, pl.ds(_GATHER_WINDOW, _GATHER_WINDOW)]],
                o_vmem.at[pl.ds(_GATHER_WINDOW, _GATHER_WINDOW)],
            )

        pltpu.emit_pipeline(
            body,
            grid=(n // (2 * _GATHER_WINDOW),),
            in_specs=[
                pl.BlockSpec((1, 2 * _GATHER_WINDOW), lambda i: (0, i)),
            ],
            out_specs=[
                pl.BlockSpec((2 * _GATHER_WINDOW, _D_PROJ), lambda i: (i, 0)),
            ],
            core_axis_name=("core", "subcore"),
            dimension_semantics=(pltpu.PARALLEL,),
        )(i_hbm, o_hbm)

    return k(table, idx_flat)


def kernel(inp, emb0, emb1, emb2, proj0, proj1, proj2):
    scale = jnp.float32(_EMB_SCALE)
    p0t = proj0.T * scale
    p1t = proj1.T * scale
    p2t = proj2.T * scale
    table = _project_tables(emb0, emb1.T, emb2.T, p0t, p1t, p2t)
    idx = _transform_idx(inp)
    idx_flat = idx.reshape(1, -1)
    n = idx_flat.shape[1]
    out = _sc_gather(table, idx_flat, n)
    return out.reshape(inp.shape + (_D_PROJ,))


# speedup vs baseline: 1.6217x; 1.0001x over previous
import jax
import jax.numpy as jnp
from jax import lax
from jax.experimental import pallas as pl
from jax.experimental.pallas import tpu as pltpu
from jax.experimental.pallas import tpu_sc as plsc

_D_PROJ = 128
_EMB_SCALE = float(_D_PROJ) ** 0.5

_BLK = 8192
_S1 = 106496
_S2 = 409600
_P_ROWS = 1015808
_B0_BLOCKS = _S1 // _BLK
_B1_BLOCKS = (_S2 - _S1) // _BLK
_B2_BLOCKS = (_P_ROWS - _S2) // _BLK

_GATHER_WINDOW = 128

_TDIMS = (((0,), (0,)), ((), ()))


def _project_tables(emb0, emb1t, emb2t, p0t, p1t, p2t):
    def body(e0_ref, e1_ref, e2_ref, p0_ref, p1_ref, p2_ref, out_ref):
        i = pl.program_id(0)

        @pl.when(i < _B0_BLOCKS)
        def _():
            out_ref[...] = jnp.dot(
                e0_ref[...], p0_ref[...], preferred_element_type=jnp.float32
            )

        @pl.when(jnp.logical_and(i >= _B0_BLOCKS, i < _B0_BLOCKS + _B1_BLOCKS))
        def _():
            out_ref[...] = lax.dot_general(
                e1_ref[...], p1_ref[...], _TDIMS,
                preferred_element_type=jnp.float32,
            )

        @pl.when(i >= _B0_BLOCKS + _B1_BLOCKS)
        def _():
            out_ref[...] = lax.dot_general(
                e2_ref[...], p2_ref[...], _TDIMS,
                preferred_element_type=jnp.float32,
            )

    return pl.pallas_call(
        body,
        grid=(_B0_BLOCKS + _B1_BLOCKS + _B2_BLOCKS,),
        in_specs=[
            pl.BlockSpec(
                (_BLK, 128), lambda i: (jnp.minimum(i, _B0_BLOCKS - 1), 0)
            ),
            pl.BlockSpec(
                (32, _BLK),
                lambda i: (0, jnp.clip(i - _B0_BLOCKS, 0, _B1_BLOCKS - 1)),
            ),
            pl.BlockSpec(
                (8, _BLK),
                lambda i: (
                    0,
                    jnp.clip(i - _B0_BLOCKS - _B1_BLOCKS, 0, _B2_BLOCKS - 1),
                ),
            ),
            pl.BlockSpec((128, 128), lambda i: (0, 0)),
            pl.BlockSpec((32, 128), lambda i: (0, 0)),
            pl.BlockSpec((8, 128), lambda i: (0, 0)),
        ],
        out_specs=pl.BlockSpec((_BLK, 128), lambda i: (i, 0)),
        out_shape=jax.ShapeDtypeStruct((_P_ROWS, _D_PROJ), jnp.float32),
        compiler_params=pltpu.CompilerParams(
            dimension_semantics=("parallel",),
        ),
    )(emb0, emb1t, emb2t, p0t, p1t, p2t)


def _transform_idx(inp):
    def body(t_ref, o_ref):
        t = t_ref[...]
        o_ref[...] = t + jnp.where(
            t < 100000, 0, jnp.where(t < 400000, _S1 - 100000, _S2 - 400000)
        )

    shape = inp.shape

    return pl.pallas_call(
        body,
        grid=(8,),
        in_specs=[pl.BlockSpec((shape[0] // 8, shape[1]), lambda i: (i, 0))],
        out_specs=pl.BlockSpec((shape[0] // 8, shape[1]), lambda i: (i, 0)),
        out_shape=jax.ShapeDtypeStruct(shape, jnp.int32),
    )(inp)


def _sc_gather(table, idx_flat, n):
    mesh = plsc.VectorSubcoreMesh(core_axis_name="core", subcore_axis_name="subcore")

    @pl.kernel(
        out_type=jax.ShapeDtypeStruct((n, _D_PROJ), jnp.float32),
        mesh=mesh,
    )
    def k(tbl_hbm, i_hbm, o_hbm):
        def body(i_vmem, o_vmem):
            for j in range(2):
                pltpu.sync_copy(
                    tbl_hbm.at[i_vmem.at[0, pl.ds(j * _GATHER_WINDOW, _GATHER_WINDOW)]],
                    o_vmem.at[pl.ds(j * _GATHER_WINDOW, _GATHER_WINDOW)],
                )

        pltpu.emit_pipeline(
            body,
            grid=(n // (2 * _GATHER_WINDOW),),
            in_specs=[
                pl.BlockSpec((1, 2 * _GATHER_WINDOW), lambda i: (0, i)),
            ],
            out_specs=[
                pl.BlockSpec((2 * _GATHER_WINDOW, _D_PROJ), lambda i: (i, 0)),
            ],
            core_axis_name=("core", "subcore"),
            dimension_semantics=(pltpu.PARALLEL,),
        )(i_hbm, o_hbm)

    return k(table, idx_flat)


def kernel(inp, emb0, emb1, emb2, proj0, proj1, proj2):
    scale = jnp.float32(_EMB_SCALE)
    p0t = proj0.T * scale
    p1t = proj1.T * scale
    p2t = proj2.T * scale
    table = _project_tables(emb0, emb1.T, emb2.T, p0t, p1t, p2t)
    idx = _transform_idx(inp)
    idx_flat = idx.reshape(1, -1)
    n = idx_flat.shape[1]
    out = _sc_gather(table, idx_flat, n)
    return out.reshape(inp.shape + (_D_PROJ,))
